# Initial kernel scaffold; baseline (speedup 1.0000x reference)
import jax
import jax.numpy as jnp
from jax.experimental import pallas as pl


def kernel(ys, yc, W1, b1, W2, b2):
    raise NotImplementedError("write your pallas kernel here")


# speedup vs baseline: 14.1496x; 14.1496x over previous
import jax
import jax.numpy as jnp
from jax.experimental import pallas as pl

_K = 5


def _gb_body(fc_ref, fs_ref, W1_ref, b1_ref, W2_ref, b2_ref, out_ref):
    f32 = jnp.float32
    fc = fc_ref[0]
    fs = fs_ref[0]
    m, F = fc.shape
    n = fs.shape[0]
    eps = f32(1e-8)

    hi = jax.lax.Precision.HIGHEST
    ones_row_F = jnp.ones((1, F), f32)
    nc_col = jnp.sum(fc * fc, axis=1, keepdims=True)
    nc_row = jax.lax.dot_general(ones_row_F, fc * fc, (((1,), (1,)), ((), ())),
                                 precision=hi, preferred_element_type=f32)
    ns_row = jax.lax.dot_general(ones_row_F, fs * fs, (((1,), (1,)), ((), ())),
                                 precision=hi, preferred_element_type=f32)

    def ncc(x, x_norm_row):
        g = jax.lax.dot_general(fc, x, (((1,), (1,)), ((), ())),
                                preferred_element_type=f32)
        return (g + eps) / (jnp.sqrt(nc_col * x_norm_row) + eps)

    def top5_sel(D):
        iota = jax.lax.broadcasted_iota(jnp.int32, D.shape, 1)
        big = jnp.int32(1 << 30)
        S = jnp.zeros(D.shape, f32)
        for _ in range(_K):
            mx = jnp.max(D, axis=1, keepdims=True)
            am = jnp.min(jnp.where(D == mx, iota, big), axis=1, keepdims=True)
            oh = iota == am
            S = S + oh.astype(f32)
            D = jnp.where(oh, -jnp.inf, D)
        return S

    S1 = top5_sel(ncc(fs, ns_row))
    S2 = top5_sel(ncc(fc, nc_row))

    ones_row_m = jnp.ones((1, m), f32)
    cnt1 = jax.lax.dot_general(ones_row_m, S1, (((1,), (0,)), ((), ())),
                               preferred_element_type=f32)
    cnt2 = jax.lax.dot_general(ones_row_m, S2, (((1,), (0,)), ((), ())),
                               preferred_element_type=f32)
    S1w = S1 * jax.lax.rsqrt(jnp.maximum(cnt1, 1.0))
    S2w = S2 * jax.lax.rsqrt(jnp.maximum(cnt2, 1.0))

    c_in = f32((2.0 * _K) ** -0.5)
    W1 = W1_ref[...]
    W2 = W2_ref[...]
    b1 = b1_ref[...]
    b2 = b2_ref[...]

    def mm(a, b):
        return jax.lax.dot_general(a, b, (((1,), (0,)), ((), ())),
                                   preferred_element_type=f32)

    agg1 = (mm(S2w, fc) + mm(S1w, fs)) * c_in
    h1 = jnp.maximum(mm(agg1, W1) + b1, 0.0)
    h1s = jnp.maximum(b1, 0.0)

    rs1 = mm(S1w, jnp.ones((n, 1), f32))
    agg2 = (mm(S2w, h1) + rs1 * h1s) * c_in
    out_ref[0] = mm(agg2, W2) + b2


def kernel(ys, yc, W1, b1, W2, b2):
    B, N1, N2, F = ys.shape
    _, C, P, _, M1, M2 = yc.shape
    n = N1 * N2
    m = M1 * M2
    fs = ys.reshape(B, n, F)
    fc = jnp.transpose(yc, (0, 4, 5, 1, 2, 3)).reshape(B, m, F)
    b1r = b1.reshape(1, F)
    b2r = b2.reshape(1, F)

    out = pl.pallas_call(
        _gb_body,
        grid=(B,),
        in_specs=[
            pl.BlockSpec((1, m, F), lambda b: (b, 0, 0)),
            pl.BlockSpec((1, n, F), lambda b: (b, 0, 0)),
            pl.BlockSpec((F, F), lambda b: (0, 0)),
            pl.BlockSpec((1, F), lambda b: (0, 0)),
            pl.BlockSpec((F, F), lambda b: (0, 0)),
            pl.BlockSpec((1, F), lambda b: (0, 0)),
        ],
        out_specs=pl.BlockSpec((1, m, F), lambda b: (b, 0, 0)),
        out_shape=jax.ShapeDtypeStruct((B, m, F), jnp.float32),
    )(fc, fs, W1, b1r, W2, b2r)

    return jnp.transpose(out, (0, 2, 1)).reshape(B, C, P, P, M1, M2)
